# initial kernel scaffold (unmeasured)
import jax
import jax.numpy as jnp
from jax import lax
from jax.experimental import pallas as pl
from jax.experimental.pallas import tpu as pltpu

N_DEV = 8


def kernel(x, w_mat, scale_x, scale_w):
    m_per, k = x.shape
    _, n = w_mat.shape
    n_per = n // N_DEV

    def body(x_ref, w_ref, sx_ref, sw_ref, out_ref, send_buf, send_sems, recv_sems):
        my = lax.axis_index("i")
        scale = sx_ref[0] * sw_ref[0]

        barrier_sem = pltpu.get_barrier_semaphore()
        for p in range(1, N_DEV):
            peer = lax.rem(my + p, N_DEV)
            pl.semaphore_signal(
                barrier_sem, inc=1,
                device_id=(peer,), device_id_type=pl.DeviceIdType.MESH,
            )
        pl.semaphore_wait(barrier_sem, N_DEV - 1)

        xv = x_ref[...].astype(jnp.float8_e4m3fn)

        def chunk(col):
            wv = w_ref[:, pl.ds(col * n_per, n_per)].astype(jnp.float8_e4m3fn)
            y = jnp.dot(xv, wv, preferred_element_type=jnp.float32) * scale
            return y * jax.nn.sigmoid(y)

        sends = []
        for p in range(1, N_DEV):
            peer = lax.rem(my + p, N_DEV)
            send_buf[p - 1] = chunk(peer)
            rdma = pltpu.make_async_remote_copy(
                src_ref=send_buf.at[p - 1],
                dst_ref=out_ref.at[pl.ds(my * m_per, m_per), :],
                send_sem=send_sems.at[p - 1],
                recv_sem=recv_sems.at[p - 1],
                device_id=(peer,),
                device_id_type=pl.DeviceIdType.MESH,
            )
            rdma.start()
            sends.append(rdma)

        out_ref[pl.ds(my * m_per, m_per), :] = chunk(my)

        for p in range(1, N_DEV):
            src = lax.rem(my + (N_DEV - p), N_DEV)
            recv = pltpu.make_async_remote_copy(
                src_ref=send_buf.at[p - 1],
                dst_ref=out_ref.at[pl.ds(src * m_per, m_per), :],
                send_sem=send_sems.at[p - 1],
                recv_sem=recv_sems.at[p - 1],
                device_id=(my,),
                device_id_type=pl.DeviceIdType.MESH,
            )
            recv.wait_recv()

        for rdma in sends:
            rdma.wait_send()

    out_shape = jax.ShapeDtypeStruct((N_DEV * m_per, n_per), jnp.float32)
    return pl.pallas_call(
        body,
        out_shape=out_shape,
        in_specs=[
            pl.BlockSpec(memory_space=pltpu.VMEM),
            pl.BlockSpec(memory_space=pltpu.VMEM),
            pl.BlockSpec(memory_space=pltpu.SMEM),
            pl.BlockSpec(memory_space=pltpu.SMEM),
        ],
        out_specs=pl.BlockSpec(memory_space=pltpu.VMEM),
        scratch_shapes=[
            pltpu.VMEM((N_DEV - 1, m_per, n_per), jnp.float32),
            pltpu.SemaphoreType.DMA((N_DEV - 1,)),
            pltpu.SemaphoreType.DMA((N_DEV - 1,)),
        ],
        compiler_params=pltpu.CompilerParams(collective_id=0),
    )(x, w_mat, scale_x, scale_w)


# baseline (device time: 59988 ns/iter reference)
import jax
import jax.numpy as jnp
from jax import lax
from jax.experimental import pallas as pl
from jax.experimental.pallas import tpu as pltpu

N_DEV = 8


def kernel(x, w_mat, scale_x, scale_w):
    m_per, k = x.shape
    _, n = w_mat.shape
    n_per = n // N_DEV

    def body(x_ref, w_ref, sx_ref, sw_ref, out_ref, send_buf, send_sems, recv_sems):
        my = lax.axis_index("i")
        scale = sx_ref[0] * sw_ref[0]

        barrier_sem = pltpu.get_barrier_semaphore()
        for p in range(1, N_DEV):
            peer = lax.rem(my + p, N_DEV)
            pl.semaphore_signal(
                barrier_sem, inc=1,
                device_id=(peer,), device_id_type=pl.DeviceIdType.MESH,
            )
        pl.semaphore_wait(barrier_sem, N_DEV - 1)

        xv = x_ref[...].astype(jnp.float8_e4m3fn)

        def chunk(col):
            wv = w_ref[:, pl.ds(col * n_per, n_per)].astype(jnp.float8_e4m3fn)
            y = jnp.dot(xv, wv, preferred_element_type=jnp.float32) * scale
            return y * jax.nn.sigmoid(y)

        sends = []
        for p in range(1, N_DEV):
            peer = lax.rem(my + p, N_DEV)
            send_buf[p - 1] = chunk(peer)
            rdma = pltpu.make_async_remote_copy(
                src_ref=send_buf.at[p - 1],
                dst_ref=out_ref.at[pl.ds(my * m_per, m_per), :],
                send_sem=send_sems.at[p - 1],
                recv_sem=recv_sems.at[p - 1],
                device_id=(peer,),
                device_id_type=pl.DeviceIdType.MESH,
            )
            rdma.start()
            sends.append(rdma)

        out_ref[pl.ds(my * m_per, m_per), :] = chunk(my)

        for p in range(1, N_DEV):
            src = lax.rem(my + (N_DEV - p), N_DEV)
            recv = pltpu.make_async_remote_copy(
                src_ref=send_buf.at[p - 1],
                dst_ref=out_ref.at[pl.ds(src * m_per, m_per), :],
                send_sem=send_sems.at[p - 1],
                recv_sem=recv_sems.at[p - 1],
                device_id=(my,),
                device_id_type=pl.DeviceIdType.MESH,
            )
            recv.wait_recv()

        for rdma in sends:
            rdma.wait_send()

    out_shape = jax.ShapeDtypeStruct((N_DEV * m_per, n_per), jnp.float32)
    return pl.pallas_call(
        body,
        out_shape=out_shape,
        in_specs=[
            pl.BlockSpec(memory_space=pltpu.VMEM),
            pl.BlockSpec(memory_space=pltpu.VMEM),
            pl.BlockSpec(memory_space=pltpu.SMEM),
            pl.BlockSpec(memory_space=pltpu.SMEM),
        ],
        out_specs=pl.BlockSpec(memory_space=pltpu.VMEM),
        scratch_shapes=[
            pltpu.VMEM((N_DEV - 1, m_per, n_per), jnp.float32),
            pltpu.SemaphoreType.DMA((N_DEV - 1,)),
            pltpu.SemaphoreType.DMA((N_DEV - 1,)),
        ],
        compiler_params=pltpu.CompilerParams(
            collective_id=0, vmem_limit_bytes=100 * 1024 * 1024
        ),
    )(x, w_mat, scale_x, scale_w)


# device time: 33984 ns/iter; 1.7652x vs baseline; 1.7652x over previous
import jax
import jax.numpy as jnp
from jax import lax
from jax.experimental import pallas as pl
from jax.experimental.pallas import tpu as pltpu

N_DEV = 8
WIRE = jnp.bfloat16
DOT = jnp.float8_e4m3fn


def kernel(x, w_mat, scale_x, scale_w):
    m_per, k = x.shape
    _, n = w_mat.shape
    n_per = n // N_DEV

    def body(x_hbm, w_hbm, sx_ref, sw_ref, out_ref,
             xbuf, wbuf, send_buf, stage, local_sems, send_sems, recv_sems):
        my = lax.axis_index("i")
        scale = sx_ref[0] * sw_ref[0]

        barrier_sem = pltpu.get_barrier_semaphore()
        for p in range(1, N_DEV):
            peer = lax.rem(my + p, N_DEV)
            pl.semaphore_signal(
                barrier_sem, inc=1,
                device_id=(peer,), device_id_type=pl.DeviceIdType.MESH,
            )

        xcp = pltpu.make_async_copy(x_hbm, xbuf, local_sems.at[0])
        xcp.start()

        def w_copy(col, slot):
            return pltpu.make_async_copy(
                w_hbm.at[:, pl.ds(col * n_per, n_per)],
                wbuf.at[slot],
                local_sems.at[1 + slot],
            )

        first_peer = lax.rem(my + 1, N_DEV)
        w_copy(first_peer, 0).start()

        xcp.wait()
        xv = xbuf[...].astype(DOT)

        def chunk(slot):
            wv = wbuf[slot].astype(DOT)
            y = jnp.dot(xv, wv, preferred_element_type=jnp.float32) * scale
            return y * jax.nn.sigmoid(y)

        sends = []
        for p in range(1, N_DEV):
            slot = (p - 1) % 2
            w_copy(lax.rem(my + p, N_DEV), slot).wait()
            nxt = lax.rem(my + p + 1, N_DEV)
            w_copy(nxt, 1 - slot).start()
            send_buf[p - 1] = chunk(slot).astype(WIRE)
            if p == 1:
                pl.semaphore_wait(barrier_sem, N_DEV - 1)
            peer = lax.rem(my + p, N_DEV)
            rdma = pltpu.make_async_remote_copy(
                src_ref=send_buf.at[p - 1],
                dst_ref=stage.at[pl.ds(my * m_per, m_per), :],
                send_sem=send_sems.at[p - 1],
                recv_sem=recv_sems.at[p - 1],
                device_id=(peer,),
                device_id_type=pl.DeviceIdType.MESH,
            )
            rdma.start()
            sends.append(rdma)

        w_copy(my, 1).wait()
        out_ref[pl.ds(my * m_per, m_per), :] = chunk(1)

        for p in range(1, N_DEV):
            src = lax.rem(my + (N_DEV - p), N_DEV)
            recv = pltpu.make_async_remote_copy(
                src_ref=send_buf.at[p - 1],
                dst_ref=stage.at[pl.ds(src * m_per, m_per), :],
                send_sem=send_sems.at[p - 1],
                recv_sem=recv_sems.at[p - 1],
                device_id=(my,),
                device_id_type=pl.DeviceIdType.MESH,
            )
            recv.wait_recv()
            out_ref[pl.ds(src * m_per, m_per), :] = (
                stage[pl.ds(src * m_per, m_per), :].astype(jnp.float32))

        for rdma in sends:
            rdma.wait_send()

    out_shape = jax.ShapeDtypeStruct((N_DEV * m_per, n_per), jnp.float32)
    return pl.pallas_call(
        body,
        out_shape=out_shape,
        in_specs=[
            pl.BlockSpec(memory_space=pl.ANY),
            pl.BlockSpec(memory_space=pl.ANY),
            pl.BlockSpec(memory_space=pltpu.SMEM),
            pl.BlockSpec(memory_space=pltpu.SMEM),
        ],
        out_specs=pl.BlockSpec(memory_space=pltpu.VMEM),
        scratch_shapes=[
            pltpu.VMEM((m_per, k), jnp.float32),
            pltpu.VMEM((2, k, n_per), jnp.float32),
            pltpu.VMEM((N_DEV - 1, m_per, n_per), WIRE),
            pltpu.VMEM((N_DEV * m_per, n_per), WIRE),
            pltpu.SemaphoreType.DMA((3,)),
            pltpu.SemaphoreType.DMA((N_DEV - 1,)),
            pltpu.SemaphoreType.DMA((N_DEV - 1,)),
        ],
        compiler_params=pltpu.CompilerParams(
            collective_id=0, vmem_limit_bytes=100 * 1024 * 1024
        ),
    )(x, w_mat, scale_x, scale_w)


# device time: 32808 ns/iter; 1.8285x vs baseline; 1.0358x over previous
import jax
import jax.numpy as jnp
from jax import lax
from jax.experimental import pallas as pl
from jax.experimental.pallas import tpu as pltpu

N_DEV = 8
WIRE = jnp.bfloat16
DOT = jnp.float8_e4m3fn


def kernel(x, w_mat, scale_x, scale_w):
    m_per, k = x.shape
    _, n = w_mat.shape
    n_per = n // N_DEV

    def body(x_hbm, w_hbm, sx_ref, sw_ref, out_ref,
             xbuf, wbuf, send_buf, stage, local_sems, send_sems, recv_sems):
        my = lax.axis_index("i")
        scale = sx_ref[0] * sw_ref[0]

        barrier_sem = pltpu.get_barrier_semaphore()
        for p in range(1, N_DEV):
            peer = lax.rem(my + p, N_DEV)
            pl.semaphore_signal(
                barrier_sem, inc=1,
                device_id=(peer,), device_id_type=pl.DeviceIdType.MESH,
            )

        xcp = pltpu.make_async_copy(x_hbm, xbuf, local_sems.at[0])
        xcp.start()

        def w_copy(p):
            col = lax.rem(my + p, N_DEV)
            return pltpu.make_async_copy(
                w_hbm.at[:, pl.ds(col * n_per, n_per)],
                wbuf.at[p - 1 if p < N_DEV else N_DEV - 1],
                local_sems.at[p],
            )

        for p in range(1, N_DEV + 1):
            w_copy(p).start()

        xcp.wait()
        xv = xbuf[...].astype(DOT)

        def chunk(slot):
            wv = wbuf[slot].astype(DOT)
            y = jnp.dot(xv, wv, preferred_element_type=jnp.float32) * scale
            return y * jax.nn.sigmoid(y)

        sends = []
        for p in range(1, N_DEV):
            w_copy(p).wait()
            send_buf[p - 1] = chunk(p - 1).astype(WIRE)
            if p == 1:
                pl.semaphore_wait(barrier_sem, N_DEV - 1)
            peer = lax.rem(my + p, N_DEV)
            rdma = pltpu.make_async_remote_copy(
                src_ref=send_buf.at[p - 1],
                dst_ref=stage.at[pl.ds(my * m_per, m_per), :],
                send_sem=send_sems.at[p - 1],
                recv_sem=recv_sems.at[p - 1],
                device_id=(peer,),
                device_id_type=pl.DeviceIdType.MESH,
            )
            rdma.start()
            sends.append(rdma)

        w_copy(N_DEV).wait()
        stage[pl.ds(my * m_per, m_per), :] = chunk(N_DEV - 1).astype(WIRE)

        for p in range(1, N_DEV):
            src = lax.rem(my + (N_DEV - p), N_DEV)
            recv = pltpu.make_async_remote_copy(
                src_ref=send_buf.at[p - 1],
                dst_ref=stage.at[pl.ds(src * m_per, m_per), :],
                send_sem=send_sems.at[p - 1],
                recv_sem=recv_sems.at[p - 1],
                device_id=(my,),
                device_id_type=pl.DeviceIdType.MESH,
            )
            recv.wait_recv()

        out_ref[...] = stage[...].astype(jnp.float32)

        for rdma in sends:
            rdma.wait_send()

    out_shape = jax.ShapeDtypeStruct((N_DEV * m_per, n_per), jnp.float32)
    return pl.pallas_call(
        body,
        out_shape=out_shape,
        in_specs=[
            pl.BlockSpec(memory_space=pl.ANY),
            pl.BlockSpec(memory_space=pl.ANY),
            pl.BlockSpec(memory_space=pltpu.SMEM),
            pl.BlockSpec(memory_space=pltpu.SMEM),
        ],
        out_specs=pl.BlockSpec(memory_space=pltpu.VMEM),
        scratch_shapes=[
            pltpu.VMEM((m_per, k), jnp.float32),
            pltpu.VMEM((N_DEV, k, n_per), jnp.float32),
            pltpu.VMEM((N_DEV - 1, m_per, n_per), WIRE),
            pltpu.VMEM((N_DEV * m_per, n_per), WIRE),
            pltpu.SemaphoreType.DMA((N_DEV + 1,)),
            pltpu.SemaphoreType.DMA((N_DEV - 1,)),
            pltpu.SemaphoreType.DMA((N_DEV - 1,)),
        ],
        compiler_params=pltpu.CompilerParams(
            collective_id=0, vmem_limit_bytes=100 * 1024 * 1024
        ),
    )(x, w_mat, scale_x, scale_w)


# device time: 31563 ns/iter; 1.9006x vs baseline; 1.0394x over previous
import jax
import jax.numpy as jnp
from jax import lax
from jax.experimental import pallas as pl
from jax.experimental.pallas import tpu as pltpu

N_DEV = 8
WIRE = jnp.bfloat16
DOT = jnp.float8_e4m3fn


def kernel(x, w_mat, scale_x, scale_w):
    m_per, k = x.shape
    _, n = w_mat.shape
    n_per = n // N_DEV

    def body(x_hbm, w_hbm, sx_ref, sw_ref, out_hbm,
             xbuf, wbuf, send_buf, stage, outtmp,
             local_sems, out_sems, send_sems, recv_sems):
        my = lax.axis_index("i")
        scale = sx_ref[0] * sw_ref[0]

        barrier_sem = pltpu.get_barrier_semaphore()
        for p in range(1, N_DEV):
            peer = lax.rem(my + p, N_DEV)
            pl.semaphore_signal(
                barrier_sem, inc=1,
                device_id=(peer,), device_id_type=pl.DeviceIdType.MESH,
            )

        xcp = pltpu.make_async_copy(x_hbm, xbuf, local_sems.at[0])
        xcp.start()

        def w_copy(p):
            col = lax.rem(my + p, N_DEV)
            return pltpu.make_async_copy(
                w_hbm.at[:, pl.ds(col * n_per, n_per)],
                wbuf.at[p - 1],
                local_sems.at[p],
            )

        for p in range(1, N_DEV + 1):
            w_copy(p).start()

        xcp.wait()
        xv = xbuf[...].astype(DOT)

        def chunk(slot):
            wv = wbuf[slot].astype(DOT)
            y = jnp.dot(xv, wv, preferred_element_type=jnp.float32) * scale
            return y * jax.nn.sigmoid(y)

        def out_block(slot, rows, value):
            outtmp[slot] = value
            ocp = pltpu.make_async_copy(
                outtmp.at[slot],
                out_hbm.at[pl.ds(rows, m_per), :],
                out_sems.at[slot],
            )
            ocp.start()
            return ocp

        sends = []
        for p in range(1, N_DEV):
            w_copy(p).wait()
            send_buf[p - 1] = chunk(p - 1).astype(WIRE)
            if p == 1:
                pl.semaphore_wait(barrier_sem, N_DEV - 1)
            peer = lax.rem(my + p, N_DEV)
            rdma = pltpu.make_async_remote_copy(
                src_ref=send_buf.at[p - 1],
                dst_ref=stage.at[pl.ds(my * m_per, m_per), :],
                send_sem=send_sems.at[p - 1],
                recv_sem=recv_sems.at[p - 1],
                device_id=(peer,),
                device_id_type=pl.DeviceIdType.MESH,
            )
            rdma.start()
            sends.append(rdma)

        w_copy(N_DEV).wait()
        out_cps = [out_block(N_DEV - 1, my * m_per, chunk(N_DEV - 1))]

        for p in range(1, N_DEV):
            src = lax.rem(my + (N_DEV - p), N_DEV)
            recv = pltpu.make_async_remote_copy(
                src_ref=send_buf.at[p - 1],
                dst_ref=stage.at[pl.ds(src * m_per, m_per), :],
                send_sem=send_sems.at[p - 1],
                recv_sem=recv_sems.at[p - 1],
                device_id=(my,),
                device_id_type=pl.DeviceIdType.MESH,
            )
            recv.wait_recv()
            out_cps.append(out_block(
                p - 1, src * m_per,
                stage[pl.ds(src * m_per, m_per), :].astype(jnp.float32)))

        for ocp in out_cps:
            ocp.wait()
        for rdma in sends:
            rdma.wait_send()

    out_shape = jax.ShapeDtypeStruct((N_DEV * m_per, n_per), jnp.float32)
    return pl.pallas_call(
        body,
        out_shape=out_shape,
        in_specs=[
            pl.BlockSpec(memory_space=pl.ANY),
            pl.BlockSpec(memory_space=pl.ANY),
            pl.BlockSpec(memory_space=pltpu.SMEM),
            pl.BlockSpec(memory_space=pltpu.SMEM),
        ],
        out_specs=pl.BlockSpec(memory_space=pl.ANY),
        scratch_shapes=[
            pltpu.VMEM((m_per, k), jnp.float32),
            pltpu.VMEM((N_DEV, k, n_per), jnp.float32),
            pltpu.VMEM((N_DEV - 1, m_per, n_per), WIRE),
            pltpu.VMEM((N_DEV * m_per, n_per), WIRE),
            pltpu.VMEM((N_DEV, m_per, n_per), jnp.float32),
            pltpu.SemaphoreType.DMA((N_DEV + 1,)),
            pltpu.SemaphoreType.DMA((N_DEV,)),
            pltpu.SemaphoreType.DMA((N_DEV - 1,)),
            pltpu.SemaphoreType.DMA((N_DEV - 1,)),
        ],
        compiler_params=pltpu.CompilerParams(
            collective_id=0, vmem_limit_bytes=100 * 1024 * 1024
        ),
    )(x, w_mat, scale_x, scale_w)


# device time: 30653 ns/iter; 1.9570x vs baseline; 1.0297x over previous
import jax
import jax.numpy as jnp
from jax import lax
from jax.experimental import pallas as pl
from jax.experimental.pallas import tpu as pltpu

N_DEV = 8
WIRE = jnp.bfloat16
DOT = jnp.float8_e4m3fn

TORDER = (6, 2, 5, 7, 1, 3, 4)


def kernel(x, w_mat, scale_x, scale_w):
    m_per, k = x.shape
    _, n = w_mat.shape
    n_per = n // N_DEV

    def body(x_hbm, w_hbm, sx_ref, sw_ref, out_hbm,
             xbuf, wbuf, send_buf, stage, outtmp,
             local_sems, out_sems, send_sems, recv_sems):
        my = lax.axis_index("i")
        scale = sx_ref[0] * sw_ref[0]

        barrier_sem = pltpu.get_barrier_semaphore()
        for t in TORDER:
            peer = lax.bitwise_xor(my, t)
            pl.semaphore_signal(
                barrier_sem, inc=1,
                device_id=(peer,), device_id_type=pl.DeviceIdType.MESH,
            )

        xcp = pltpu.make_async_copy(x_hbm, xbuf, local_sems.at[0])
        xcp.start()

        def w_copy(p):
            col = my if p == N_DEV else lax.bitwise_xor(my, TORDER[p - 1])
            return pltpu.make_async_copy(
                w_hbm.at[:, pl.ds(col * n_per, n_per)],
                wbuf.at[p - 1],
                local_sems.at[p],
            )

        for p in range(1, N_DEV + 1):
            w_copy(p).start()

        xcp.wait()
        xv = xbuf[...].astype(DOT)

        def chunk(slot):
            wv = wbuf[slot].astype(DOT)
            y = jnp.dot(xv, wv, preferred_element_type=jnp.float32) * scale
            return y * jax.nn.sigmoid(y)

        def out_block(slot, rows, value):
            outtmp[slot] = value
            ocp = pltpu.make_async_copy(
                outtmp.at[slot],
                out_hbm.at[pl.ds(rows, m_per), :],
                out_sems.at[slot],
            )
            ocp.start()
            return ocp

        sends = []
        for p in range(1, N_DEV):
            w_copy(p).wait()
            send_buf[p - 1] = chunk(p - 1).astype(WIRE)
            if p == 1:
                pl.semaphore_wait(barrier_sem, N_DEV - 1)
            peer = lax.bitwise_xor(my, TORDER[p - 1])
            rdma = pltpu.make_async_remote_copy(
                src_ref=send_buf.at[p - 1],
                dst_ref=stage.at[pl.ds(my * m_per, m_per), :],
                send_sem=send_sems.at[p - 1],
                recv_sem=recv_sems.at[p - 1],
                device_id=(peer,),
                device_id_type=pl.DeviceIdType.MESH,
            )
            rdma.start()
            sends.append(rdma)

        w_copy(N_DEV).wait()
        out_cps = [out_block(N_DEV - 1, my * m_per, chunk(N_DEV - 1))]

        for p in range(1, N_DEV):
            src = lax.bitwise_xor(my, TORDER[p - 1])
            recv = pltpu.make_async_remote_copy(
                src_ref=send_buf.at[p - 1],
                dst_ref=stage.at[pl.ds(src * m_per, m_per), :],
                send_sem=send_sems.at[p - 1],
                recv_sem=recv_sems.at[p - 1],
                device_id=(my,),
                device_id_type=pl.DeviceIdType.MESH,
            )
            recv.wait_recv()
            out_cps.append(out_block(
                p - 1, src * m_per,
                stage[pl.ds(src * m_per, m_per), :].astype(jnp.float32)))

        for ocp in out_cps:
            ocp.wait()
        for rdma in sends:
            rdma.wait_send()

    out_shape = jax.ShapeDtypeStruct((N_DEV * m_per, n_per), jnp.float32)
    return pl.pallas_call(
        body,
        out_shape=out_shape,
        in_specs=[
            pl.BlockSpec(memory_space=pl.ANY),
            pl.BlockSpec(memory_space=pl.ANY),
            pl.BlockSpec(memory_space=pltpu.SMEM),
            pl.BlockSpec(memory_space=pltpu.SMEM),
        ],
        out_specs=pl.BlockSpec(memory_space=pl.ANY),
        scratch_shapes=[
            pltpu.VMEM((m_per, k), jnp.float32),
            pltpu.VMEM((N_DEV, k, n_per), jnp.float32),
            pltpu.VMEM((N_DEV - 1, m_per, n_per), WIRE),
            pltpu.VMEM((N_DEV * m_per, n_per), WIRE),
            pltpu.VMEM((N_DEV, m_per, n_per), jnp.float32),
            pltpu.SemaphoreType.DMA((N_DEV + 1,)),
            pltpu.SemaphoreType.DMA((N_DEV,)),
            pltpu.SemaphoreType.DMA((N_DEV - 1,)),
            pltpu.SemaphoreType.DMA((N_DEV - 1,)),
        ],
        compiler_params=pltpu.CompilerParams(
            collective_id=0, vmem_limit_bytes=100 * 1024 * 1024
        ),
    )(x, w_mat, scale_x, scale_w)
